# baseline (speedup 1.0000x reference)
import functools

import jax
import jax.numpy as jnp
from jax import lax
from jax.experimental import pallas as pl
from jax.experimental.pallas import tpu as pltpu
from jax.experimental.pallas import tpu_sc as plsc

N = 10000
E = 320000
HID = 32

NC, NS = 2, 16
NW = NC * NS
NP = 10240
EPW = 10240
CHUNK = 512
NCH = EPW // CHUNK
EPW0 = E // NW
CH0 = 1000
NCH0 = EPW0 // CH0
CHG = 500
NCHG = EPW0 // CHG

_sc_mesh = plsc.VectorSubcoreMesh(core_axis_name="c", subcore_axis_name="s",
                                  num_cores=NC, num_subcores=NS)
_sc_params = pltpu.CompilerParams(use_tc_tiling_on_sc=False,
                                  needs_layout_passes=False)


@functools.partial(
    pl.kernel,
    out_type=jax.ShapeDtypeStruct((NC, NP, HID), jnp.float32),
    mesh=_sc_mesh,
    compiler_params=_sc_params,
    scratch_types=[
        pltpu.VMEM((NCH, CHUNK), jnp.int32),
        pltpu.VMEM((NCH, CHUNK), jnp.int32),
        pltpu.VMEM((CHUNK, HID), jnp.float32),
        pltpu.VMEM((CHUNK, HID), jnp.float32),
        pltpu.VMEM_SHARED((NP, HID), jnp.float32),
        pltpu.SemaphoreType.DMA,
        pltpu.SemaphoreType.DMA,
    ],
)
def _sc_rowscatter(src_hbm, dst_hbm, tbl_hbm, zeros_hbm, out_hbm,
                   idx_s, idx_d, rowbuf0, rowbuf1, acc, sem0, sem1):
    cid = lax.axis_index("c")
    sid = lax.axis_index("s")
    wid = sid * NC + cid
    rows = NP // NS
    pltpu.sync_copy(zeros_hbm.at[pl.ds(sid * rows, rows)],
                    acc.at[pl.ds(sid * rows, rows)])
    pltpu.sync_copy(src_hbm.at[wid], idx_s)
    pltpu.sync_copy(dst_hbm.at[wid], idx_d)
    plsc.subcore_barrier()
    bufs = (rowbuf0, rowbuf1)
    sems = (sem0, sem1)

    def fire(g):
        return pltpu.async_copy(tbl_hbm.at[idx_s.at[g]], bufs[g % 2],
                                sems[g % 2])

    h = fire(0)
    for g in range(NCH):
        h_next = fire(g + 1) if g + 1 < NCH else None
        h.wait()
        pltpu.sync_copy(bufs[g % 2], acc.at[idx_d.at[g]], add=True)
        h = h_next
    plsc.subcore_barrier()
    pltpu.sync_copy(acc.at[pl.ds(sid * rows, rows)],
                    out_hbm.at[cid, pl.ds(sid * rows, rows)])


@functools.partial(
    pl.kernel,
    out_type=(jax.ShapeDtypeStruct((NC, 2 * NP, HID), jnp.float32),
              jax.ShapeDtypeStruct((NC, 2 * NP), jnp.float32)),
    mesh=_sc_mesh,
    compiler_params=_sc_params,
    scratch_types=[
        pltpu.VMEM((NCH, CHUNK), jnp.int32),
        pltpu.VMEM((NCH, CHUNK), jnp.int32),
        pltpu.VMEM((NP,), jnp.float32),
        pltpu.VMEM((NP,), jnp.float32),
        pltpu.VMEM((16,), jnp.float32),
        pltpu.VMEM((2, CHUNK), jnp.int32),
        pltpu.VMEM((2, CHUNK), jnp.int32),
        pltpu.VMEM((2, CHUNK), jnp.float32),
        pltpu.VMEM((CHUNK, HID), jnp.float32),
        pltpu.VMEM((CHUNK, HID), jnp.float32),
        pltpu.VMEM_SHARED((2 * NP, HID), jnp.float32),
        pltpu.VMEM_SHARED((2 * NP,), jnp.float32),
        pltpu.SemaphoreType.DMA,
        pltpu.SemaphoreType.DMA,
    ],
)
def _sc_gat(src_hbm, dst_hbm, u_hbm, v_hbm, th_hbm, t_hbm, zrows_hbm, zvec_hbm,
            outn_hbm, outd_hbm,
            idx_s, idx_d, utbl, vtbl, thbuf, gbuf, sbuf, vbuf,
            rowbuf0, rowbuf1, accn, accd, sem0, sem1):
    cid = lax.axis_index("c")
    sid = lax.axis_index("s")
    wid = sid * NC + cid
    rows = 2 * NP // NS
    pltpu.sync_copy(zrows_hbm.at[pl.ds(0, rows)],
                    accn.at[pl.ds(sid * rows, rows)])
    pltpu.sync_copy(zvec_hbm.at[pl.ds(sid * rows, rows)],
                    accd.at[pl.ds(sid * rows, rows)])
    pltpu.sync_copy(src_hbm.at[wid], idx_s)
    pltpu.sync_copy(dst_hbm.at[wid], idx_d)
    pltpu.sync_copy(u_hbm, utbl)
    pltpu.sync_copy(v_hbm, vtbl)
    pltpu.sync_copy(th_hbm, thbuf)
    plsc.subcore_barrier()
    th = thbuf[...]
    rowbufs = (rowbuf0, rowbuf1)
    sems = (sem0, sem1)

    def compute_idx(g):
        def body(i, _):
            sv = idx_s[g, pl.ds(i * 16, 16)]
            dv = idx_d[g, pl.ds(i * 16, 16)]
            u = plsc.load_gather(utbl, [sv])
            v = plsc.load_gather(vtbl, [dv])
            cls = (u + v) < th
            val = jnp.exp(jnp.where(cls, 0.2, 1.0) * u)
            off = jnp.where(cls, NP, 0).astype(jnp.int32)
            gbuf[g % 2, pl.ds(i * 16, 16)] = sv + off
            sbuf[g % 2, pl.ds(i * 16, 16)] = dv + off
            vbuf[g % 2, pl.ds(i * 16, 16)] = val
            return 0
        lax.fori_loop(0, CHUNK // 16, body, 0)

    def fire(g):
        return pltpu.async_copy(t_hbm.at[gbuf.at[g % 2]], rowbufs[g % 2],
                                sems[g % 2])

    compute_idx(0)
    h = fire(0)
    for g in range(NCH):
        if g + 1 < NCH:
            compute_idx(g + 1)
            h_next = fire(g + 1)
        else:
            h_next = None
        h.wait()
        pltpu.sync_copy(rowbufs[g % 2], accn.at[sbuf.at[g % 2]], add=True)
        pltpu.sync_copy(vbuf.at[g % 2], accd.at[sbuf.at[g % 2]], add=True)
        h = h_next
    plsc.subcore_barrier()
    pltpu.sync_copy(accn.at[pl.ds(sid * rows, rows)],
                    outn_hbm.at[cid, pl.ds(sid * rows, rows)])
    pltpu.sync_copy(accd.at[pl.ds(sid * rows, rows)],
                    outd_hbm.at[cid, pl.ds(sid * rows, rows)])


@functools.partial(
    pl.kernel,
    out_type=(jax.ShapeDtypeStruct((NW, EPW0, 32), jnp.float32),
              jax.ShapeDtypeStruct((NW, EPW0, 32), jnp.float32)),
    mesh=_sc_mesh,
    compiler_params=_sc_params,
    scratch_types=[
        pltpu.VMEM((NCHG, CHG), jnp.int32),
        pltpu.VMEM((NCHG, CHG), jnp.int32),
        pltpu.VMEM((CHG, 32), jnp.float32),
        pltpu.VMEM((CHG, 32), jnp.float32),
        pltpu.VMEM((CHG, 32), jnp.float32),
        pltpu.VMEM((CHG, 32), jnp.float32),
        pltpu.SemaphoreType.DMA,
        pltpu.SemaphoreType.DMA,
        pltpu.SemaphoreType.DMA,
        pltpu.SemaphoreType.DMA,
    ],
)
def _sc_cg_gather(src_hbm, dst_hbm, ts_hbm, td_hbm, gs_hbm, gd_hbm,
                  idx_s, idx_d, bs0, bd0, bs1, bd1, gsem0, gsem1,
                  ssem0, ssem1):
    cid = lax.axis_index("c")
    sid = lax.axis_index("s")
    wid = sid * NC + cid
    pltpu.sync_copy(src_hbm.at[wid], idx_s)
    pltpu.sync_copy(dst_hbm.at[wid], idx_d)
    bufs = ((bs0, bd0), (bs1, bd1))
    gsems = (gsem0, gsem1)
    ssems = (ssem0, ssem1)

    def fire(g):
        bs, bd = bufs[g % 2]
        return (pltpu.async_copy(ts_hbm.at[idx_s.at[g]], bs, gsems[g % 2]),
                pltpu.async_copy(td_hbm.at[idx_d.at[g]], bd, gsems[g % 2]))

    def store(g):
        bs, bd = bufs[g % 2]
        sl = pl.ds(g * CHG, CHG)
        return (pltpu.async_copy(bs, gs_hbm.at[wid, sl], ssems[g % 2]),
                pltpu.async_copy(bd, gd_hbm.at[wid, sl], ssems[g % 2]))

    store_h = [None, None]
    gh = fire(0)
    for g in range(NCHG):
        if g + 1 < NCHG:
            slot = (g + 1) % 2
            if store_h[slot] is not None:
                for h in store_h[slot]:
                    h.wait()
                store_h[slot] = None
            gh_next = fire(g + 1)
        else:
            gh_next = None
        for h in gh:
            h.wait()
        store_h[g % 2] = store(g)
        gh = gh_next
    for hs in store_h:
        if hs is not None:
            for h in hs:
                h.wait()


@functools.partial(
    pl.kernel,
    out_type=jax.ShapeDtypeStruct((NC, NP, 16), jnp.float32),
    mesh=_sc_mesh,
    compiler_params=_sc_params,
    scratch_types=[
        pltpu.VMEM((NCH0, CH0), jnp.int32),
        pltpu.VMEM((CH0, 16), jnp.float32),
        pltpu.VMEM((CH0, 16), jnp.float32),
        pltpu.VMEM_SHARED((NP, 16), jnp.float32),
        pltpu.SemaphoreType.DMA,
        pltpu.SemaphoreType.DMA,
    ],
)
def _sc_cg_scatter(dst_hbm, msg_hbm, zeros_hbm, out_hbm, idx_d, bufm0, bufm1,
                   acc, sem0, sem1):
    cid = lax.axis_index("c")
    sid = lax.axis_index("s")
    wid = sid * NC + cid
    rows = NP // NS
    pltpu.sync_copy(zeros_hbm.at[pl.ds(sid * rows, rows)],
                    acc.at[pl.ds(sid * rows, rows)])
    pltpu.sync_copy(dst_hbm.at[wid], idx_d)
    plsc.subcore_barrier()
    bufs = (bufm0, bufm1)
    sems = (sem0, sem1)

    def fire(g):
        return pltpu.async_copy(msg_hbm.at[wid, pl.ds(g * CH0, CH0)],
                                bufs[g % 2], sems[g % 2])

    h = fire(0)
    for g in range(NCH0):
        h_next = fire(g + 1) if g + 1 < NCH0 else None
        h.wait()
        pltpu.sync_copy(bufs[g % 2], acc.at[idx_d.at[g]], add=True)
        h = h_next
    plsc.subcore_barrier()
    pltpu.sync_copy(acc.at[pl.ds(sid * rows, rows)],
                    out_hbm.at[cid, pl.ds(sid * rows, rows)])


def _msg_body(ea_ref, gs_ref, gd_ref, wf_ref, ws_ref, bf_ref, bs_ref,
              mask_ref, one_ref, o_ref):
    ea = ea_ref[...]
    g = gs_ref[0] + gd_ref[0]
    zf = g[:, :16] + jnp.dot(ea, wf_ref[...],
                             preferred_element_type=jnp.float32) + bf_ref[...]
    zs = g[:, 16:] + jnp.dot(ea, ws_ref[...],
                             preferred_element_type=jnp.float32) + bs_ref[...]
    sig = 1.0 / (1.0 + jnp.exp(-zf))
    sp = jnp.maximum(zs, 0.0) + jnp.log1p(jnp.exp(-jnp.abs(zs)))
    o_ref[0] = (sig * sp) * mask_ref[...] + one_ref[...]


_MSG_BLK = 2000
_MSG_NB = EPW0 // _MSG_BLK


def _msg_pass(ea, gsrc, gdst, wf_e, ws_e, bf, bs):
    mask = jnp.zeros((1, 16), jnp.float32).at[0, :13].set(1.0)
    one = jnp.zeros((1, 16), jnp.float32).at[0, 13].set(1.0)
    bf_p = jnp.zeros((1, 16), jnp.float32).at[0, :13].set(bf)
    bs_p = jnp.zeros((1, 16), jnp.float32).at[0, :13].set(bs)
    wf_p = jnp.zeros((44, 16), jnp.float32).at[:, :13].set(wf_e)
    ws_p = jnp.zeros((44, 16), jnp.float32).at[:, :13].set(ws_e)
    grid = (NW, _MSG_NB)
    return pl.pallas_call(
        _msg_body,
        grid=grid,
        in_specs=[
            pl.BlockSpec((_MSG_BLK, 44), lambda i, j: (i * _MSG_NB + j, 0)),
            pl.BlockSpec((1, _MSG_BLK, 32), lambda i, j: (i, j, 0)),
            pl.BlockSpec((1, _MSG_BLK, 32), lambda i, j: (i, j, 0)),
            pl.BlockSpec((44, 16), lambda i, j: (0, 0)),
            pl.BlockSpec((44, 16), lambda i, j: (0, 0)),
            pl.BlockSpec((1, 16), lambda i, j: (0, 0)),
            pl.BlockSpec((1, 16), lambda i, j: (0, 0)),
            pl.BlockSpec((1, 16), lambda i, j: (0, 0)),
            pl.BlockSpec((1, 16), lambda i, j: (0, 0)),
        ],
        out_specs=pl.BlockSpec((1, _MSG_BLK, 16), lambda i, j: (i, j, 0)),
        out_shape=jax.ShapeDtypeStruct((NW, EPW0, 16), jnp.float32),
    )(ea, gsrc, gdst, wf_p, ws_p, bf_p, bs_p, mask, one)


def _bn(x, g, b):
    mu = jnp.mean(x, axis=0)
    var = jnp.var(x, axis=0)
    return g * (x - mu) / jnp.sqrt(var + 1e-5) + b


def _pad_rows(a, rows):
    return jnp.zeros((rows,) + a.shape[1:], a.dtype).at[:a.shape[0]].set(a)


def _gat(x, srcp, dstp, W, a_src, a_dst, b, zrows, zvec):
    h = x @ W
    asrc = h @ a_src
    adst = h @ a_dst
    A = jnp.max(asrc)
    B = jnp.max(adst)
    ab = A + B
    Mg = jnp.maximum(ab, 0.0)
    u = asrc - A
    v = adst - B
    T = jnp.concatenate([
        _pad_rows(jnp.exp(u)[:, None] * h, NP),
        _pad_rows(jnp.exp(0.2 * u)[:, None] * h, NP),
    ], axis=0)
    u_p = _pad_rows(u, NP)
    v_p = _pad_rows(v, NP)
    theta = jnp.full((16,), -ab, jnp.float32)
    pn, pd = _sc_gat(srcp, dstp, u_p, v_p, theta, T, zrows, zvec)
    d1 = jnp.exp(v + (ab - Mg))
    d2 = jnp.exp(0.2 * v + (0.2 * ab - Mg))
    num = (d1[:, None] * (pn[0, :N] + pn[1, :N])
           + d2[:, None] * (pn[0, NP:NP + N] + pn[1, NP:NP + N]))
    den = (d1 * (pd[0, :N] + pd[1, :N])
           + d2 * (pd[0, NP:NP + N] + pd[1, NP:NP + N]))
    wl = jnp.exp(jax.nn.leaky_relu(asrc + adst, 0.2) - Mg)
    num = num + wl[:, None] * h
    den = den + wl
    return num / den[:, None] + b


def _tail_kernel(xc_ref, w0_ref, b0_ref, w1_ref, b1_ref, o_ref):
    h = jnp.dot(xc_ref[...], w0_ref[...], preferred_element_type=jnp.float32)
    h = h + b0_ref[...]
    h = jnp.dot(h, w1_ref[...], preferred_element_type=jnp.float32)
    h = h + b1_ref[...]
    o_ref[...] = jnp.maximum(h, 0.0)[:, 0]


def _tail(xc, Wf00, bf00, Wf0, bf0):
    return pl.pallas_call(
        _tail_kernel,
        out_shape=jax.ShapeDtypeStruct((N,), jnp.float32),
    )(xc, Wf00, bf00[None, :], Wf0, bf0[None, :])


def _pad_edges(idx):
    pad = jnp.full((NW, EPW - EPW0), N, jnp.int32)
    return jnp.concatenate([idx.reshape(NW, EPW0), pad], axis=1).reshape(NW, NCH, CHUNK)


def kernel(x, edge_index, edgesAttr, W1, a_src1, a_dst1, b1, W2, a_src2, a_dst2, b2,
           W3, a_src3, a_dst3, b3, Wf, bf, Ws, bs, W6, b6, g1, be1, g2, be2,
           g3, be3, g4, be4, g6, be6, Wf00, bf00, Wf0, bf0):
    src = edge_index[0].astype(jnp.int32)
    dst = edge_index[1].astype(jnp.int32)
    srcp = _pad_edges(src)
    dstp = _pad_edges(dst)
    dst0 = dst.reshape(NW, NCH0, CH0)
    zrows = jnp.zeros((NP, HID), jnp.float32)
    zvec = jnp.zeros((2 * NP,), jnp.float32)
    z16 = jnp.zeros((NP, 16), jnp.float32)

    x13 = x[:, :13]
    ts = (jnp.zeros((NP, 32), jnp.float32)
          .at[:N, :13].set(x13 @ Wf[13:26])
          .at[:N, 16:29].set(x13 @ Ws[13:26]))
    td = (jnp.zeros((NP, 32), jnp.float32)
          .at[:N, :13].set(x13 @ Wf[:13])
          .at[:N, 16:29].set(x13 @ Ws[:13]))
    srcg = src.reshape(NW, NCHG, CHG)
    dstg = dst.reshape(NW, NCHG, CHG)
    gsrc, gdst = _sc_cg_gather(srcg, dstg, ts, td)
    msg = _msg_pass(edgesAttr, gsrc, gdst, Wf[26:], Ws[26:], bf, bs)
    cg = _sc_cg_scatter(dst0, msg, z16)
    cgacc = cg[0, :N] + cg[1, :N]
    ce = x13 + cgacc[:, :13]
    deg = cgacc[:, 13] + 1.0
    xe = _bn(jax.nn.relu(ce), g4, be4)

    c1 = _gat(x, srcp, dstp, W1, a_src1, a_dst1, b1, zrows, zvec)
    x1 = _bn(jax.nn.relu(c1), g1, be1)
    c2 = _gat(x1, srcp, dstp, W2, a_src2, a_dst2, b2, zrows, zvec)
    x2 = _bn(jax.nn.relu(c2), g2, be2)
    c3 = _gat(x2, srcp, dstp, W3, a_src3, a_dst3, b3, zrows, zvec)
    x3 = _bn(jax.nn.relu(c3), g3, be3)

    dinv = jax.lax.rsqrt(deg)
    h6 = xe @ W6
    hh = dinv[:, None] * h6
    part = _sc_rowscatter(srcp, dstp, _pad_rows(hh, NP), zrows)
    acc6 = part[0, :N] + part[1, :N]
    c6 = dinv[:, None] * acc6 + dinv[:, None] * hh + b6
    x6 = _bn(jax.nn.relu(c6), g6, be6)

    xc = jnp.concatenate([x1, x2, x3, xe, x6], axis=-1)
    return _tail(xc, Wf00, bf00, Wf0, bf0)


# speedup vs baseline: 1.0671x; 1.0671x over previous
import functools

import jax
import jax.numpy as jnp
from jax import lax
from jax.experimental import pallas as pl
from jax.experimental.pallas import tpu as pltpu
from jax.experimental.pallas import tpu_sc as plsc

N = 10000
E = 320000
HID = 32

NC, NS = 2, 16
NW = NC * NS
NP = 10240
EPW = 10240
CHUNK = 512
NCH = EPW // CHUNK
EPW0 = E // NW
CH0 = 1000
NCH0 = EPW0 // CH0
CHG = 500
NCHG = EPW0 // CHG

_sc_mesh = plsc.VectorSubcoreMesh(core_axis_name="c", subcore_axis_name="s",
                                  num_cores=NC, num_subcores=NS)
_sc_params = pltpu.CompilerParams(use_tc_tiling_on_sc=False,
                                  needs_layout_passes=False)


@functools.partial(
    pl.kernel,
    out_type=jax.ShapeDtypeStruct((NC, NP, HID), jnp.float32),
    mesh=_sc_mesh,
    compiler_params=_sc_params,
    scratch_types=[
        pltpu.VMEM((NCH, CHUNK), jnp.int32),
        pltpu.VMEM((NCH, CHUNK), jnp.int32),
        pltpu.VMEM((CHUNK, HID), jnp.float32),
        pltpu.VMEM((CHUNK, HID), jnp.float32),
        pltpu.VMEM_SHARED((NP, HID), jnp.float32),
        pltpu.SemaphoreType.DMA,
        pltpu.SemaphoreType.DMA,
    ],
)
def _sc_rowscatter(src_hbm, dst_hbm, tbl_hbm, zeros_hbm, out_hbm,
                   idx_s, idx_d, rowbuf0, rowbuf1, acc, sem0, sem1):
    cid = lax.axis_index("c")
    sid = lax.axis_index("s")
    wid = sid * NC + cid
    rows = NP // NS
    pltpu.sync_copy(zeros_hbm.at[pl.ds(sid * rows, rows)],
                    acc.at[pl.ds(sid * rows, rows)])
    pltpu.sync_copy(src_hbm.at[wid], idx_s)
    pltpu.sync_copy(dst_hbm.at[wid], idx_d)
    plsc.subcore_barrier()
    bufs = (rowbuf0, rowbuf1)
    sems = (sem0, sem1)

    def fire(g):
        return pltpu.async_copy(tbl_hbm.at[idx_s.at[g]], bufs[g % 2],
                                sems[g % 2])

    h = fire(0)
    for g in range(NCH):
        h_next = fire(g + 1) if g + 1 < NCH else None
        h.wait()
        pltpu.sync_copy(bufs[g % 2], acc.at[idx_d.at[g]], add=True)
        h = h_next
    plsc.subcore_barrier()
    pltpu.sync_copy(acc.at[pl.ds(sid * rows, rows)],
                    out_hbm.at[cid, pl.ds(sid * rows, rows)])


@functools.partial(
    pl.kernel,
    out_type=(jax.ShapeDtypeStruct((NC, 2 * NP, HID), jnp.float32),
              jax.ShapeDtypeStruct((NC, 2 * NP), jnp.float32)),
    mesh=_sc_mesh,
    compiler_params=_sc_params,
    scratch_types=[
        pltpu.VMEM((NCH, CHUNK), jnp.int32),
        pltpu.VMEM((NCH, CHUNK), jnp.int32),
        pltpu.VMEM((NP,), jnp.float32),
        pltpu.VMEM((NP,), jnp.float32),
        pltpu.VMEM((16,), jnp.float32),
        pltpu.VMEM((2, CHUNK), jnp.int32),
        pltpu.VMEM((2, CHUNK), jnp.int32),
        pltpu.VMEM((2, CHUNK), jnp.float32),
        pltpu.VMEM((CHUNK, HID), jnp.float32),
        pltpu.VMEM((CHUNK, HID), jnp.float32),
        pltpu.VMEM_SHARED((2 * NP, HID), jnp.float32),
        pltpu.VMEM_SHARED((2 * NP,), jnp.float32),
        pltpu.SemaphoreType.DMA,
        pltpu.SemaphoreType.DMA,
    ],
)
def _sc_gat(src_hbm, dst_hbm, u_hbm, v_hbm, th_hbm, t_hbm, zrows_hbm, zvec_hbm,
            outn_hbm, outd_hbm,
            idx_s, idx_d, utbl, vtbl, thbuf, gbuf, sbuf, vbuf,
            rowbuf0, rowbuf1, accn, accd, sem0, sem1):
    cid = lax.axis_index("c")
    sid = lax.axis_index("s")
    wid = sid * NC + cid
    rows = 2 * NP // NS
    pltpu.sync_copy(zrows_hbm.at[pl.ds(0, rows)],
                    accn.at[pl.ds(sid * rows, rows)])
    pltpu.sync_copy(zvec_hbm.at[pl.ds(sid * rows, rows)],
                    accd.at[pl.ds(sid * rows, rows)])
    pltpu.sync_copy(src_hbm.at[wid], idx_s)
    pltpu.sync_copy(dst_hbm.at[wid], idx_d)
    pltpu.sync_copy(u_hbm, utbl)
    pltpu.sync_copy(v_hbm, vtbl)
    pltpu.sync_copy(th_hbm, thbuf)
    plsc.subcore_barrier()
    th = thbuf[...]
    rowbufs = (rowbuf0, rowbuf1)
    sems = (sem0, sem1)

    def compute_idx(g):
        def body(i, _):
            sv = idx_s[g, pl.ds(i * 16, 16)]
            dv = idx_d[g, pl.ds(i * 16, 16)]
            u = plsc.load_gather(utbl, [sv])
            v = plsc.load_gather(vtbl, [dv])
            cls = (u + v) < th
            val = jnp.exp(jnp.where(cls, 0.2, 1.0) * u)
            off = jnp.where(cls, NP, 0).astype(jnp.int32)
            gbuf[g % 2, pl.ds(i * 16, 16)] = sv + off
            sbuf[g % 2, pl.ds(i * 16, 16)] = dv + off
            vbuf[g % 2, pl.ds(i * 16, 16)] = val
            return 0
        lax.fori_loop(0, CHUNK // 16, body, 0)

    def fire(g):
        return pltpu.async_copy(t_hbm.at[gbuf.at[g % 2]], rowbufs[g % 2],
                                sems[g % 2])

    compute_idx(0)
    h = fire(0)
    for g in range(NCH):
        if g + 1 < NCH:
            compute_idx(g + 1)
            h_next = fire(g + 1)
        else:
            h_next = None
        h.wait()
        pltpu.sync_copy(rowbufs[g % 2], accn.at[sbuf.at[g % 2]], add=True)
        pltpu.sync_copy(vbuf.at[g % 2], accd.at[sbuf.at[g % 2]], add=True)
        h = h_next
    plsc.subcore_barrier()
    pltpu.sync_copy(accn.at[pl.ds(sid * rows, rows)],
                    outn_hbm.at[cid, pl.ds(sid * rows, rows)])
    pltpu.sync_copy(accd.at[pl.ds(sid * rows, rows)],
                    outd_hbm.at[cid, pl.ds(sid * rows, rows)])


@functools.partial(
    pl.kernel,
    out_type=(jax.ShapeDtypeStruct((NW, EPW0, 16), jnp.float32),
              jax.ShapeDtypeStruct((NW, EPW0, 16), jnp.float32)),
    mesh=_sc_mesh,
    compiler_params=_sc_params,
    scratch_types=[
        pltpu.VMEM((NCHG, CHG), jnp.int32),
        pltpu.VMEM((NCHG, CHG), jnp.int32),
        pltpu.VMEM((CHG, 16), jnp.float32),
        pltpu.VMEM((CHG, 16), jnp.float32),
        pltpu.VMEM((CHG, 16), jnp.float32),
        pltpu.VMEM((CHG, 16), jnp.float32),
        pltpu.SemaphoreType.DMA,
        pltpu.SemaphoreType.DMA,
        pltpu.SemaphoreType.DMA,
        pltpu.SemaphoreType.DMA,
    ],
)
def _sc_cg_gather(src_hbm, dst_hbm, ts_hbm, td_hbm, gs_hbm, gd_hbm,
                  idx_s, idx_d, bs0, bd0, bs1, bd1, gsem0, gsem1,
                  ssem0, ssem1):
    cid = lax.axis_index("c")
    sid = lax.axis_index("s")
    wid = sid * NC + cid
    pltpu.sync_copy(src_hbm.at[wid], idx_s)
    pltpu.sync_copy(dst_hbm.at[wid], idx_d)
    bufs = ((bs0, bd0), (bs1, bd1))
    gsems = (gsem0, gsem1)
    ssems = (ssem0, ssem1)

    def fire(g):
        bs, bd = bufs[g % 2]
        return (pltpu.async_copy(ts_hbm.at[idx_s.at[g]], bs, gsems[g % 2]),
                pltpu.async_copy(td_hbm.at[idx_d.at[g]], bd, gsems[g % 2]))

    def store(g):
        bs, bd = bufs[g % 2]
        sl = pl.ds(g * CHG, CHG)
        return (pltpu.async_copy(bs, gs_hbm.at[wid, sl], ssems[g % 2]),
                pltpu.async_copy(bd, gd_hbm.at[wid, sl], ssems[g % 2]))

    store_h = [None, None]
    gh = fire(0)
    for g in range(NCHG):
        if g + 1 < NCHG:
            slot = (g + 1) % 2
            if store_h[slot] is not None:
                for h in store_h[slot]:
                    h.wait()
                store_h[slot] = None
            gh_next = fire(g + 1)
        else:
            gh_next = None
        for h in gh:
            h.wait()
        store_h[g % 2] = store(g)
        gh = gh_next
    for hs in store_h:
        if hs is not None:
            for h in hs:
                h.wait()


@functools.partial(
    pl.kernel,
    out_type=jax.ShapeDtypeStruct((NC, NP, 16), jnp.float32),
    mesh=_sc_mesh,
    compiler_params=_sc_params,
    scratch_types=[
        pltpu.VMEM((NCH0, CH0), jnp.int32),
        pltpu.VMEM((CH0, 16), jnp.float32),
        pltpu.VMEM((CH0, 16), jnp.float32),
        pltpu.VMEM_SHARED((NP, 16), jnp.float32),
        pltpu.SemaphoreType.DMA,
        pltpu.SemaphoreType.DMA,
    ],
)
def _sc_cg_scatter(dst_hbm, msg_hbm, zeros_hbm, out_hbm, idx_d, bufm0, bufm1,
                   acc, sem0, sem1):
    cid = lax.axis_index("c")
    sid = lax.axis_index("s")
    wid = sid * NC + cid
    rows = NP // NS
    pltpu.sync_copy(zeros_hbm.at[pl.ds(sid * rows, rows)],
                    acc.at[pl.ds(sid * rows, rows)])
    pltpu.sync_copy(dst_hbm.at[wid], idx_d)
    plsc.subcore_barrier()
    bufs = (bufm0, bufm1)
    sems = (sem0, sem1)

    def fire(g):
        return pltpu.async_copy(msg_hbm.at[wid, pl.ds(g * CH0, CH0)],
                                bufs[g % 2], sems[g % 2])

    h = fire(0)
    for g in range(NCH0):
        h_next = fire(g + 1) if g + 1 < NCH0 else None
        h.wait()
        pltpu.sync_copy(bufs[g % 2], acc.at[idx_d.at[g]], add=True)
        h = h_next
    plsc.subcore_barrier()
    pltpu.sync_copy(acc.at[pl.ds(sid * rows, rows)],
                    out_hbm.at[cid, pl.ds(sid * rows, rows)])


def _msg_body(ea_ref, gs_ref, gd_ref, wfd_ref, wfs_ref, wsd_ref, wss_ref,
              wfe_ref, wse_ref, bf_ref, bs_ref, mask_ref, one_ref, o_ref):
    ea = ea_ref[...]
    gs = gs_ref[0]
    gd = gd_ref[0]
    zf = (jnp.dot(gd, wfd_ref[...], preferred_element_type=jnp.float32)
          + jnp.dot(gs, wfs_ref[...], preferred_element_type=jnp.float32)
          + jnp.dot(ea, wfe_ref[...], preferred_element_type=jnp.float32)
          + bf_ref[...])
    zs = (jnp.dot(gd, wsd_ref[...], preferred_element_type=jnp.float32)
          + jnp.dot(gs, wss_ref[...], preferred_element_type=jnp.float32)
          + jnp.dot(ea, wse_ref[...], preferred_element_type=jnp.float32)
          + bs_ref[...])
    sig = 1.0 / (1.0 + jnp.exp(-zf))
    sp = jnp.maximum(zs, 0.0) + jnp.log1p(jnp.exp(-jnp.abs(zs)))
    o_ref[0] = (sig * sp) * mask_ref[...] + one_ref[...]


_MSG_BLK = 2000
_MSG_NB = EPW0 // _MSG_BLK


def _msg_pass(ea, gsrc, gdst, Wf_all, Ws_all, bf, bs):
    mask = jnp.zeros((1, 16), jnp.float32).at[0, :13].set(1.0)
    one = jnp.zeros((1, 16), jnp.float32).at[0, 13].set(1.0)
    bf_p = jnp.zeros((1, 16), jnp.float32).at[0, :13].set(bf)
    bs_p = jnp.zeros((1, 16), jnp.float32).at[0, :13].set(bs)
    wfd = jnp.zeros((16, 16), jnp.float32).at[:13, :13].set(Wf_all[:13])
    wfs = jnp.zeros((16, 16), jnp.float32).at[:13, :13].set(Wf_all[13:26])
    wsd = jnp.zeros((16, 16), jnp.float32).at[:13, :13].set(Ws_all[:13])
    wss = jnp.zeros((16, 16), jnp.float32).at[:13, :13].set(Ws_all[13:26])
    wfe = jnp.zeros((44, 16), jnp.float32).at[:, :13].set(Wf_all[26:])
    wse = jnp.zeros((44, 16), jnp.float32).at[:, :13].set(Ws_all[26:])
    grid = (NW, _MSG_NB)
    return pl.pallas_call(
        _msg_body,
        grid=grid,
        in_specs=[
            pl.BlockSpec((_MSG_BLK, 44), lambda i, j: (i * _MSG_NB + j, 0)),
            pl.BlockSpec((1, _MSG_BLK, 16), lambda i, j: (i, j, 0)),
            pl.BlockSpec((1, _MSG_BLK, 16), lambda i, j: (i, j, 0)),
            pl.BlockSpec((16, 16), lambda i, j: (0, 0)),
            pl.BlockSpec((16, 16), lambda i, j: (0, 0)),
            pl.BlockSpec((16, 16), lambda i, j: (0, 0)),
            pl.BlockSpec((16, 16), lambda i, j: (0, 0)),
            pl.BlockSpec((44, 16), lambda i, j: (0, 0)),
            pl.BlockSpec((44, 16), lambda i, j: (0, 0)),
            pl.BlockSpec((1, 16), lambda i, j: (0, 0)),
            pl.BlockSpec((1, 16), lambda i, j: (0, 0)),
            pl.BlockSpec((1, 16), lambda i, j: (0, 0)),
            pl.BlockSpec((1, 16), lambda i, j: (0, 0)),
        ],
        out_specs=pl.BlockSpec((1, _MSG_BLK, 16), lambda i, j: (i, j, 0)),
        out_shape=jax.ShapeDtypeStruct((NW, EPW0, 16), jnp.float32),
    )(ea, gsrc, gdst, wfd, wfs, wsd, wss, wfe, wse, bf_p, bs_p, mask, one)


def _bn(x, g, b):
    mu = jnp.mean(x, axis=0)
    var = jnp.var(x, axis=0)
    return g * (x - mu) / jnp.sqrt(var + 1e-5) + b


def _pad_rows(a, rows):
    return jnp.zeros((rows,) + a.shape[1:], a.dtype).at[:a.shape[0]].set(a)


def _gat(x, srcp, dstp, W, a_src, a_dst, b, zrows, zvec):
    h = x @ W
    asrc = h @ a_src
    adst = h @ a_dst
    A = jnp.max(asrc)
    B = jnp.max(adst)
    ab = A + B
    Mg = jnp.maximum(ab, 0.0)
    u = asrc - A
    v = adst - B
    T = jnp.concatenate([
        _pad_rows(jnp.exp(u)[:, None] * h, NP),
        _pad_rows(jnp.exp(0.2 * u)[:, None] * h, NP),
    ], axis=0)
    u_p = _pad_rows(u, NP)
    v_p = _pad_rows(v, NP)
    theta = jnp.full((16,), -ab, jnp.float32)
    pn, pd = _sc_gat(srcp, dstp, u_p, v_p, theta, T, zrows, zvec)
    d1 = jnp.exp(v + (ab - Mg))
    d2 = jnp.exp(0.2 * v + (0.2 * ab - Mg))
    num = (d1[:, None] * (pn[0, :N] + pn[1, :N])
           + d2[:, None] * (pn[0, NP:NP + N] + pn[1, NP:NP + N]))
    den = (d1 * (pd[0, :N] + pd[1, :N])
           + d2 * (pd[0, NP:NP + N] + pd[1, NP:NP + N]))
    wl = jnp.exp(jax.nn.leaky_relu(asrc + adst, 0.2) - Mg)
    num = num + wl[:, None] * h
    den = den + wl
    return num / den[:, None] + b


def _tail_kernel(xc_ref, w0_ref, b0_ref, w1_ref, b1_ref, o_ref):
    h = jnp.dot(xc_ref[...], w0_ref[...], preferred_element_type=jnp.float32)
    h = h + b0_ref[...]
    h = jnp.dot(h, w1_ref[...], preferred_element_type=jnp.float32)
    h = h + b1_ref[...]
    o_ref[...] = jnp.maximum(h, 0.0)[:, 0]


def _tail(xc, Wf00, bf00, Wf0, bf0):
    return pl.pallas_call(
        _tail_kernel,
        out_shape=jax.ShapeDtypeStruct((N,), jnp.float32),
    )(xc, Wf00, bf00[None, :], Wf0, bf0[None, :])


def _pad_edges(idx):
    pad = jnp.full((NW, EPW - EPW0), N, jnp.int32)
    return jnp.concatenate([idx.reshape(NW, EPW0), pad], axis=1).reshape(NW, NCH, CHUNK)


def kernel(x, edge_index, edgesAttr, W1, a_src1, a_dst1, b1, W2, a_src2, a_dst2, b2,
           W3, a_src3, a_dst3, b3, Wf, bf, Ws, bs, W6, b6, g1, be1, g2, be2,
           g3, be3, g4, be4, g6, be6, Wf00, bf00, Wf0, bf0):
    src = edge_index[0].astype(jnp.int32)
    dst = edge_index[1].astype(jnp.int32)
    srcp = _pad_edges(src)
    dstp = _pad_edges(dst)
    dst0 = dst.reshape(NW, NCH0, CH0)
    zrows = jnp.zeros((NP, HID), jnp.float32)
    zvec = jnp.zeros((2 * NP,), jnp.float32)
    z16 = jnp.zeros((NP, 16), jnp.float32)

    x13 = x[:, :13]
    tx = jnp.zeros((NP, 16), jnp.float32).at[:N, :13].set(x13)
    srcg = src.reshape(NW, NCHG, CHG)
    dstg = dst.reshape(NW, NCHG, CHG)
    gsrc, gdst = _sc_cg_gather(srcg, dstg, tx, tx)
    msg = _msg_pass(edgesAttr, gsrc, gdst, Wf, Ws, bf, bs)
    cg = _sc_cg_scatter(dst0, msg, z16)
    cgacc = cg[0, :N] + cg[1, :N]
    ce = x13 + cgacc[:, :13]
    deg = cgacc[:, 13] + 1.0
    xe = _bn(jax.nn.relu(ce), g4, be4)

    c1 = _gat(x, srcp, dstp, W1, a_src1, a_dst1, b1, zrows, zvec)
    x1 = _bn(jax.nn.relu(c1), g1, be1)
    c2 = _gat(x1, srcp, dstp, W2, a_src2, a_dst2, b2, zrows, zvec)
    x2 = _bn(jax.nn.relu(c2), g2, be2)
    c3 = _gat(x2, srcp, dstp, W3, a_src3, a_dst3, b3, zrows, zvec)
    x3 = _bn(jax.nn.relu(c3), g3, be3)

    dinv = jax.lax.rsqrt(deg)
    h6 = xe @ W6
    hh = dinv[:, None] * h6
    part = _sc_rowscatter(srcp, dstp, _pad_rows(hh, NP), zrows)
    acc6 = part[0, :N] + part[1, :N]
    c6 = dinv[:, None] * acc6 + dinv[:, None] * hh + b6
    x6 = _bn(jax.nn.relu(c6), g6, be6)

    xc = jnp.concatenate([x1, x2, x3, xe, x6], axis=-1)
    return _tail(xc, Wf00, bf00, Wf0, bf0)
